# baseline (speedup 1.0000x reference)
import functools

import jax
import jax.numpy as jnp
from jax import lax
from jax.experimental import pallas as pl
from jax.experimental.pallas import tpu as pltpu
from jax.experimental.pallas import tpu_sc as plsc

D = 2048
T = 2048
B = 2
N = 8192
EPS = 1e-8

NC = 2
NS = 16
NW = NC * NS
L = 16

GROUPS = N // L
GPW = GROUPS // NW
PPW = GPW * L
NSEG = 32
TSEG = T // NSEG
UNROLL = 8
CUT = 23.03
SLAB = 64
BT = 256


def _sc_body(zt_hbm, r_hbm, ii_hbm, jj_hbm, num_hbm, s_hbm,
             ii_all, jj_all, r_all, ridx,
             rows, num0_st, num1_st, s_st, sem):
  wid = lax.axis_index("s") * NC + lax.axis_index("c")
  lanes = lax.iota(jnp.int32, L)
  base = wid * PPW
  pltpu.sync_copy(ii_hbm.at[pl.ds(base, PPW)], ii_all)
  pltpu.sync_copy(jj_hbm.at[pl.ds(base, PPW)], jj_all)
  pltpu.sync_copy(r_hbm.at[pl.ds(base, PPW)], r_all)

  def group_body(gl, carry0):
    ii = ii_all[pl.ds(gl * L, L)]
    jj = jj_all[pl.ds(gl * L, L)]
    r_v = r_all[pl.ds(gl * L, L)]
    d = jnp.exp(-r_v)
    r_min = jnp.min(r_v)
    lanes_f = lanes.astype(jnp.float32)
    step = r_min * float(TSEG)
    n_segs = jnp.sum((lanes_f * step < CUT).astype(jnp.int32))
    n_segs = n_segs + jnp.sum(((lanes_f + float(L)) * step < CUT).astype(jnp.int32))

    for b in range(B):
      row_i = (ii + b * D) * NSEG
      row_j = (jj + b * D) * NSEG

      def seg_body(s, seg_carry):
        w, acc, ssum = seg_carry
        ridx[pl.ds(0, L)] = row_i + (NSEG - 1 - s)
        ridx[pl.ds(L, L)] = row_j + (NSEG - 1 - s)
        pltpu.async_copy(zt_hbm.at[ridx], rows, sem).wait()

        def t_chunk(c, ch_carry):
          w, acc, ssum, tvec = ch_carry
          for _ in range(UNROLL):
            zi = plsc.load_gather(rows, [lanes, tvec])
            zj = plsc.load_gather(rows, [lanes + L, tvec])
            acc = acc + zi * zj * w
            ssum = ssum + w
            w = w * d
            tvec = tvec - 1
          return w, acc, ssum, tvec

        init = (w, acc, ssum, jnp.full((L,), TSEG - 1, jnp.int32))
        res = lax.fori_loop(0, TSEG // UNROLL, t_chunk, init)
        return res[0], res[1], res[2]

      init = (jnp.ones((L,), jnp.float32),
              jnp.zeros((L,), jnp.float32),
              jnp.zeros((L,), jnp.float32))
      _, acc, ssum = lax.fori_loop(0, n_segs, seg_body, init)

      if b == 0:
        num0_st[pl.ds(gl * L, L)] = acc
        s_st[pl.ds(gl * L, L)] = ssum
      else:
        num1_st[pl.ds(gl * L, L)] = acc
    return carry0

  lax.fori_loop(0, GPW, group_body, None)

  pltpu.sync_copy(num0_st, num_hbm.at[0, pl.ds(base, PPW)])
  pltpu.sync_copy(num1_st, num_hbm.at[1, pl.ds(base, PPW)])
  pltpu.sync_copy(s_st, s_hbm.at[pl.ds(base, PPW)])


_sc_call = functools.partial(
    pl.kernel,
    mesh=plsc.VectorSubcoreMesh(core_axis_name="c", subcore_axis_name="s"),
    compiler_params=pltpu.CompilerParams(
        use_tc_tiling_on_sc=False, needs_layout_passes=False),
    out_type=[jax.ShapeDtypeStruct((B, N), jnp.float32),
              jax.ShapeDtypeStruct((N,), jnp.float32)],
    scratch_types=[
        pltpu.VMEM((PPW,), jnp.int32),
        pltpu.VMEM((PPW,), jnp.int32),
        pltpu.VMEM((PPW,), jnp.float32),
        pltpu.VMEM((2 * L,), jnp.int32),
        pltpu.VMEM((2 * L, TSEG), jnp.float32),
        pltpu.VMEM((PPW,), jnp.float32),
        pltpu.VMEM((PPW,), jnp.float32),
        pltpu.VMEM((PPW,), jnp.float32),
        pltpu.SemaphoreType.DMA,
    ],
)(_sc_body)


def _general(z_hist, r, ii, jj):
  zt = jnp.transpose(z_hist, (0, 2, 1)).reshape(B * D * NSEG, TSEG)
  num, s = _sc_call(zt, r, ii, jj)
  return num / jnp.sqrt(s + EPS)[None, :]


def _gram_body(r0_ref, zs_ref, zs2_ref, c_ref):
  bi = lax.broadcasted_iota(jnp.int32, (SLAB, BT), 0).astype(jnp.float32)
  w = jnp.exp(-r0_ref[0, 0] * (float(SLAB - 1) - bi))
  a = zs_ref[0] * w
  c_ref[0] = lax.dot_general(a, zs2_ref[0], (((0,), (0,)), ((), ())),
                             precision=lax.Precision.HIGHEST,
                             preferred_element_type=jnp.float32)


_gram_call = pl.pallas_call(
    _gram_body,
    grid=(B, D // BT, D // BT),
    in_specs=[
        pl.BlockSpec(memory_space=pltpu.SMEM),
        pl.BlockSpec((1, SLAB, BT), lambda b, i, j: (b, T // SLAB - 1, i)),
        pl.BlockSpec((1, SLAB, BT), lambda b, i, j: (b, T // SLAB - 1, j)),
    ],
    out_specs=pl.BlockSpec((1, BT, BT), lambda b, i, j: (b, i, j)),
    out_shape=jax.ShapeDtypeStruct((B, D, D), jnp.float32),
)


def _scg_body(cflat_hbm, r_hbm, ii_hbm, jj_hbm, num_hbm, s_hbm,
              ii_all, jj_all, r_all,
              eidx0, eidx1, eidx2, eidx3,
              vals0, vals1, vals2, vals3,
              num0_st, num1_st, s_st, sem):
  wid = lax.axis_index("s") * NC + lax.axis_index("c")
  base = wid * PPW
  pltpu.sync_copy(ii_hbm.at[pl.ds(base, PPW)], ii_all)
  pltpu.sync_copy(jj_hbm.at[pl.ds(base, PPW)], jj_all)
  pltpu.sync_copy(r_hbm.at[pl.ds(base, PPW)], r_all)

  eidx = (eidx0, eidx1, eidx2, eidx3)
  vals = (vals0, vals1, vals2, vals3)
  for c in range(4):
    for k in range(4):
      gl = c * 4 + k
      e = ii_all[pl.ds(gl * L, L)] * D + jj_all[pl.ds(gl * L, L)]
      eidx[c][pl.ds(k * 2 * L, L)] = e
      eidx[c][pl.ds(k * 2 * L + L, L)] = e + D * D
  copies = [pltpu.async_copy(cflat_hbm.at[eidx[c]], vals[c], sem)
            for c in range(4)]
  for cp in copies:
    cp.wait()

  for c in range(4):
    for k in range(4):
      gl = c * 4 + k
      num0_st[pl.ds(gl * L, L)] = vals[c][pl.ds(k * 2 * L, L)]
      num1_st[pl.ds(gl * L, L)] = vals[c][pl.ds(k * 2 * L + L, L)]
      q = jnp.exp(-r_all[pl.ds(gl * L, L)])
      ssum = 1.0 + q
      qq = q * q
      for _ in range(10):
        ssum = ssum * (1.0 + qq)
        qq = qq * qq
      s_st[pl.ds(gl * L, L)] = ssum

  pltpu.sync_copy(num0_st, num_hbm.at[0, pl.ds(base, PPW)])
  pltpu.sync_copy(num1_st, num_hbm.at[1, pl.ds(base, PPW)])
  pltpu.sync_copy(s_st, s_hbm.at[pl.ds(base, PPW)])


_scg_call = functools.partial(
    pl.kernel,
    mesh=plsc.VectorSubcoreMesh(core_axis_name="c", subcore_axis_name="s"),
    compiler_params=pltpu.CompilerParams(
        use_tc_tiling_on_sc=False, needs_layout_passes=False),
    out_type=[jax.ShapeDtypeStruct((B, N), jnp.float32),
              jax.ShapeDtypeStruct((N,), jnp.float32)],
    scratch_types=(
        [pltpu.VMEM((PPW,), jnp.int32),
         pltpu.VMEM((PPW,), jnp.int32),
         pltpu.VMEM((PPW,), jnp.float32)]
        + [pltpu.VMEM((8 * L,), jnp.int32) for _ in range(4)]
        + [pltpu.VMEM((8 * L,), jnp.float32) for _ in range(4)]
        + [pltpu.VMEM((PPW,), jnp.float32) for _ in range(3)]
        + [pltpu.SemaphoreType.DMA]
    ),
)(_scg_body)


def _fast(z_hist, r, ii, jj):
  cmat = _gram_call(r[:1].reshape(1, 1), z_hist, z_hist)
  num, s = _scg_call(cmat.reshape(B * D * D), r, ii, jj)
  return num / jnp.sqrt(s + EPS)[None, :]


@jax.jit
def kernel(z_hist, decay_rates, idx_i, idx_j):
  r = jax.nn.softplus(decay_rates)
  ii = idx_i.astype(jnp.int32)
  jj = idx_j.astype(jnp.int32)
  uniform = jnp.all(decay_rates == decay_rates[0])
  slab_ok = r[0] * float(SLAB) >= CUT
  return lax.cond(jnp.logical_and(uniform, slab_ok),
                  lambda: _fast(z_hist, r, ii, jj),
                  lambda: _general(z_hist, r, ii, jj))


# speedup vs baseline: 1.3670x; 1.3670x over previous
import functools

import jax
import jax.numpy as jnp
from jax import lax
from jax.experimental import pallas as pl
from jax.experimental.pallas import tpu as pltpu
from jax.experimental.pallas import tpu_sc as plsc

D = 2048
T = 2048
B = 2
N = 8192
EPS = 1e-8

NC = 2
NS = 16
NW = NC * NS
L = 16

GROUPS = N // L
GPW = GROUPS // NW
PPW = GPW * L
TSEG = 64
NSEG = T // TSEG
UNROLL = 8
CUT = 23.03


def _make_sc_body(nseg):
  def _sc_body(zt_hbm, r_hbm, ii_hbm, jj_hbm, num_hbm, s_hbm,
               ii_all, jj_all, r_all, ridx,
               rows, num0_st, num1_st, s_st, sem):
    wid = lax.axis_index("s") * NC + lax.axis_index("c")
    lanes = lax.iota(jnp.int32, L)
    base = wid * PPW
    pltpu.sync_copy(ii_hbm.at[pl.ds(base, PPW)], ii_all)
    pltpu.sync_copy(jj_hbm.at[pl.ds(base, PPW)], jj_all)
    pltpu.sync_copy(r_hbm.at[pl.ds(base, PPW)], r_all)

    def group_body(gl, carry0):
      ii = ii_all[pl.ds(gl * L, L)]
      jj = jj_all[pl.ds(gl * L, L)]
      r_v = r_all[pl.ds(gl * L, L)]
      d = jnp.exp(-r_v)
      r_min = jnp.min(r_v)
      lanes_f = lanes.astype(jnp.float32)
      step = r_min * float(TSEG)
      n_segs = jnp.sum((lanes_f * step < CUT).astype(jnp.int32))
      if nseg > L:
        n_segs = n_segs + jnp.sum(
            ((lanes_f + float(L)) * step < CUT).astype(jnp.int32))
      n_segs = jnp.minimum(n_segs, nseg)

      for b in range(B):
        row_i = (ii + b * D) * nseg
        row_j = (jj + b * D) * nseg

        def seg_body(s, seg_carry):
          w, acc, ssum = seg_carry
          ridx[pl.ds(0, L)] = row_i + (nseg - 1 - s)
          ridx[pl.ds(L, L)] = row_j + (nseg - 1 - s)
          pltpu.async_copy(zt_hbm.at[ridx], rows, sem).wait()

          def t_chunk(c, ch_carry):
            w, acc, ssum, tvec = ch_carry
            for _ in range(UNROLL):
              zi = plsc.load_gather(rows, [lanes, tvec])
              zj = plsc.load_gather(rows, [lanes + L, tvec])
              acc = acc + zi * zj * w
              ssum = ssum + w
              w = w * d
              tvec = tvec - 1
            return w, acc, ssum, tvec

          init = (w, acc, ssum, jnp.full((L,), TSEG - 1, jnp.int32))
          res = lax.fori_loop(0, TSEG // UNROLL, t_chunk, init)
          return res[0], res[1], res[2]

        init = (jnp.ones((L,), jnp.float32),
                jnp.zeros((L,), jnp.float32),
                jnp.zeros((L,), jnp.float32))
        _, acc, ssum = lax.fori_loop(0, n_segs, seg_body, init)

        if b == 0:
          num0_st[pl.ds(gl * L, L)] = acc
          s_st[pl.ds(gl * L, L)] = ssum
        else:
          num1_st[pl.ds(gl * L, L)] = acc
      return carry0

    lax.fori_loop(0, GPW, group_body, None)

    pltpu.sync_copy(num0_st, num_hbm.at[0, pl.ds(base, PPW)])
    pltpu.sync_copy(num1_st, num_hbm.at[1, pl.ds(base, PPW)])
    pltpu.sync_copy(s_st, s_hbm.at[pl.ds(base, PPW)])

  return _sc_body


def _make_sc_call(nseg):
  return functools.partial(
      pl.kernel,
      mesh=plsc.VectorSubcoreMesh(core_axis_name="c", subcore_axis_name="s"),
      compiler_params=pltpu.CompilerParams(
          use_tc_tiling_on_sc=False, needs_layout_passes=False),
      out_type=[jax.ShapeDtypeStruct((B, N), jnp.float32),
                jax.ShapeDtypeStruct((N,), jnp.float32)],
      scratch_types=[
          pltpu.VMEM((PPW,), jnp.int32),
          pltpu.VMEM((PPW,), jnp.int32),
          pltpu.VMEM((PPW,), jnp.float32),
          pltpu.VMEM((2 * L,), jnp.int32),
          pltpu.VMEM((2 * L, TSEG), jnp.float32),
          pltpu.VMEM((PPW,), jnp.float32),
          pltpu.VMEM((PPW,), jnp.float32),
          pltpu.VMEM((PPW,), jnp.float32),
          pltpu.SemaphoreType.DMA,
      ],
  )(_make_sc_body(nseg))


_sc_call_general = _make_sc_call(NSEG)
_sc_call_fast = _make_sc_call(1)


def _general(z_hist, r, ii, jj):
  zt = jnp.transpose(z_hist, (0, 2, 1)).reshape(B * D * NSEG, TSEG)
  num, s = _sc_call_general(zt, r, ii, jj)
  return num / jnp.sqrt(s + EPS)[None, :]


def _fast(z_hist, r, ii, jj):
  zt = jnp.transpose(z_hist[:, T - TSEG:, :], (0, 2, 1)).reshape(B * D, TSEG)
  num, s = _sc_call_fast(zt, r, ii, jj)
  return num / jnp.sqrt(s + EPS)[None, :]


@jax.jit
def kernel(z_hist, decay_rates, idx_i, idx_j):
  r = jax.nn.softplus(decay_rates)
  ii = idx_i.astype(jnp.int32)
  jj = idx_j.astype(jnp.int32)
  slab_ok = jnp.min(r) * float(TSEG) >= CUT
  return lax.cond(slab_ok,
                  lambda: _fast(z_hist, r, ii, jj),
                  lambda: _general(z_hist, r, ii, jj))


# speedup vs baseline: 1.6394x; 1.1993x over previous
import functools

import jax
import jax.numpy as jnp
from jax import lax
from jax.experimental import pallas as pl
from jax.experimental.pallas import tpu as pltpu
from jax.experimental.pallas import tpu_sc as plsc

D = 2048
T = 2048
B = 2
N = 8192
EPS = 1e-8

NC = 2
NS = 16
NW = NC * NS
L = 16

GROUPS = N // L
GPW = GROUPS // NW
PPW = GPW * L
TSEG = 64
NSEG = T // TSEG
UNROLL = 8
CUT = 23.03


def _make_sc_body(nseg):
  def _sc_body(zt_hbm, r_hbm, ii_hbm, jj_hbm, num_hbm, s_hbm,
               ii_all, jj_all, r_all, ridx,
               rows, num0_st, num1_st, s_st, sem):
    wid = lax.axis_index("s") * NC + lax.axis_index("c")
    lanes = lax.iota(jnp.int32, L)
    base = wid * PPW
    pltpu.sync_copy(ii_hbm.at[pl.ds(base, PPW)], ii_all)
    pltpu.sync_copy(jj_hbm.at[pl.ds(base, PPW)], jj_all)
    pltpu.sync_copy(r_hbm.at[pl.ds(base, PPW)], r_all)

    def group_body(gl, carry0):
      ii = ii_all[pl.ds(gl * L, L)]
      jj = jj_all[pl.ds(gl * L, L)]
      r_v = r_all[pl.ds(gl * L, L)]
      d = jnp.exp(-r_v)
      r_min = jnp.min(r_v)
      lanes_f = lanes.astype(jnp.float32)
      step = r_min * float(TSEG)
      n_segs = jnp.sum((lanes_f * step < CUT).astype(jnp.int32))
      if nseg > L:
        n_segs = n_segs + jnp.sum(
            ((lanes_f + float(L)) * step < CUT).astype(jnp.int32))
      n_segs = jnp.minimum(n_segs, nseg)

      for b in range(B):
        row_i = (ii + b * D) * nseg
        row_j = (jj + b * D) * nseg

        def seg_body(s, seg_carry):
          w, acc, ssum = seg_carry
          ridx[pl.ds(0, L)] = row_i + (nseg - 1 - s)
          ridx[pl.ds(L, L)] = row_j + (nseg - 1 - s)
          pltpu.async_copy(zt_hbm.at[ridx], rows, sem).wait()

          def t_chunk(c, ch_carry):
            w, acc, ssum, tvec = ch_carry
            for _ in range(UNROLL):
              zi = plsc.load_gather(rows, [lanes, tvec])
              zj = plsc.load_gather(rows, [lanes + L, tvec])
              acc = acc + zi * zj * w
              ssum = ssum + w
              w = w * d
              tvec = tvec - 1
            return w, acc, ssum, tvec

          init = (w, acc, ssum, jnp.full((L,), TSEG - 1, jnp.int32))
          res = lax.fori_loop(0, TSEG // UNROLL, t_chunk, init)
          return res[0], res[1], res[2]

        init = (jnp.ones((L,), jnp.float32),
                jnp.zeros((L,), jnp.float32),
                jnp.zeros((L,), jnp.float32))
        _, acc, ssum = lax.fori_loop(0, n_segs, seg_body, init)

        if b == 0:
          num0_st[pl.ds(gl * L, L)] = acc
          s_st[pl.ds(gl * L, L)] = ssum
        else:
          num1_st[pl.ds(gl * L, L)] = acc
      return carry0

    lax.fori_loop(0, GPW, group_body, None)

    pltpu.sync_copy(num0_st, num_hbm.at[0, pl.ds(base, PPW)])
    pltpu.sync_copy(num1_st, num_hbm.at[1, pl.ds(base, PPW)])
    pltpu.sync_copy(s_st, s_hbm.at[pl.ds(base, PPW)])

  return _sc_body


def _make_sc_call(nseg):
  return functools.partial(
      pl.kernel,
      mesh=plsc.VectorSubcoreMesh(core_axis_name="c", subcore_axis_name="s"),
      compiler_params=pltpu.CompilerParams(
          use_tc_tiling_on_sc=False, needs_layout_passes=False),
      out_type=[jax.ShapeDtypeStruct((B, N), jnp.float32),
                jax.ShapeDtypeStruct((N,), jnp.float32)],
      scratch_types=[
          pltpu.VMEM((PPW,), jnp.int32),
          pltpu.VMEM((PPW,), jnp.int32),
          pltpu.VMEM((PPW,), jnp.float32),
          pltpu.VMEM((2 * L,), jnp.int32),
          pltpu.VMEM((2 * L, TSEG), jnp.float32),
          pltpu.VMEM((PPW,), jnp.float32),
          pltpu.VMEM((PPW,), jnp.float32),
          pltpu.VMEM((PPW,), jnp.float32),
          pltpu.SemaphoreType.DMA,
      ],
  )(_make_sc_body(nseg))


_sc_call_general = _make_sc_call(NSEG)


def _fast_body(zt_hbm, r_hbm, ii_hbm, jj_hbm, num_hbm, s_hbm,
               ii_all, jj_all, r_all, ridx0, ridx1, rows0, rows1,
               num0_st, num1_st, s_st, sem0, sem1):
  wid = lax.axis_index("s") * NC + lax.axis_index("c")
  lanes = lax.iota(jnp.int32, L)
  base = wid * PPW
  pltpu.sync_copy(ii_hbm.at[pl.ds(base, PPW)], ii_all)
  pltpu.sync_copy(jj_hbm.at[pl.ds(base, PPW)], jj_all)
  pltpu.sync_copy(r_hbm.at[pl.ds(base, PPW)], r_all)

  ridx = (ridx0, ridx1)
  rows = (rows0, rows1)
  sems = (sem0, sem1)
  NU = GPW * B

  def issue(u, slot):
    gl, b = u // B, u % B
    ii = ii_all[pl.ds(gl * L, L)]
    jj = jj_all[pl.ds(gl * L, L)]
    ridx[slot][pl.ds(0, L)] = ii + b * D
    ridx[slot][pl.ds(L, L)] = jj + b * D
    return pltpu.async_copy(zt_hbm.at[ridx[slot]], rows[slot], sems[slot])

  def compute(u, slot):
    gl, b = u // B, u % B
    d = jnp.exp(-r_all[pl.ds(gl * L, L)])
    rw = rows[slot]

    def t_chunk(c, ch_carry):
      w, acc, ssum, tvec = ch_carry
      for _ in range(UNROLL):
        zi = plsc.load_gather(rw, [lanes, tvec])
        zj = plsc.load_gather(rw, [lanes + L, tvec])
        acc = acc + zi * zj * w
        ssum = ssum + w
        w = w * d
        tvec = tvec - 1
      return w, acc, ssum, tvec

    init = (jnp.ones((L,), jnp.float32),
            jnp.zeros((L,), jnp.float32),
            jnp.zeros((L,), jnp.float32),
            jnp.full((L,), TSEG - 1, jnp.int32))
    res = lax.fori_loop(0, TSEG // UNROLL, t_chunk, init)
    acc, ssum = res[1], res[2]
    if b == 0:
      num0_st[pl.ds(gl * L, L)] = acc
      s_st[pl.ds(gl * L, L)] = ssum
    else:
      num1_st[pl.ds(gl * L, L)] = acc

  h = issue(0, 0)
  for u in range(NU):
    slot = u % 2
    nh = issue(u + 1, 1 - slot) if u + 1 < NU else None
    h.wait()
    compute(u, slot)
    h = nh

  pltpu.sync_copy(num0_st, num_hbm.at[0, pl.ds(base, PPW)])
  pltpu.sync_copy(num1_st, num_hbm.at[1, pl.ds(base, PPW)])
  pltpu.sync_copy(s_st, s_hbm.at[pl.ds(base, PPW)])


_sc_call_fast = functools.partial(
    pl.kernel,
    mesh=plsc.VectorSubcoreMesh(core_axis_name="c", subcore_axis_name="s"),
    compiler_params=pltpu.CompilerParams(
        use_tc_tiling_on_sc=False, needs_layout_passes=False),
    out_type=[jax.ShapeDtypeStruct((B, N), jnp.float32),
              jax.ShapeDtypeStruct((N,), jnp.float32)],
    scratch_types=[
        pltpu.VMEM((PPW,), jnp.int32),
        pltpu.VMEM((PPW,), jnp.int32),
        pltpu.VMEM((PPW,), jnp.float32),
        pltpu.VMEM((2 * L,), jnp.int32),
        pltpu.VMEM((2 * L,), jnp.int32),
        pltpu.VMEM((2 * L, TSEG), jnp.float32),
        pltpu.VMEM((2 * L, TSEG), jnp.float32),
        pltpu.VMEM((PPW,), jnp.float32),
        pltpu.VMEM((PPW,), jnp.float32),
        pltpu.VMEM((PPW,), jnp.float32),
        pltpu.SemaphoreType.DMA,
        pltpu.SemaphoreType.DMA,
    ],
)(_fast_body)


def _general(z_hist, r, ii, jj):
  zt = jnp.transpose(z_hist, (0, 2, 1)).reshape(B * D * NSEG, TSEG)
  num, s = _sc_call_general(zt, r, ii, jj)
  return num / jnp.sqrt(s + EPS)[None, :]


def _fast(z_hist, r, ii, jj):
  zt = jnp.transpose(z_hist[:, T - TSEG:, :], (0, 2, 1)).reshape(B * D, TSEG)
  num, s = _sc_call_fast(zt, r, ii, jj)
  return num / jnp.sqrt(s + EPS)[None, :]


@jax.jit
def kernel(z_hist, decay_rates, idx_i, idx_j):
  r = jax.nn.softplus(decay_rates)
  ii = idx_i.astype(jnp.int32)
  jj = idx_j.astype(jnp.int32)
  slab_ok = jnp.min(r) * float(TSEG) >= CUT
  return lax.cond(slab_ok,
                  lambda: _fast(z_hist, r, ii, jj),
                  lambda: _general(z_hist, r, ii, jj))


# speedup vs baseline: 1.8848x; 1.1497x over previous
import functools

import jax
import jax.numpy as jnp
from jax import lax
from jax.experimental import pallas as pl
from jax.experimental.pallas import tpu as pltpu
from jax.experimental.pallas import tpu_sc as plsc

D = 2048
T = 2048
B = 2
N = 8192
EPS = 1e-8

NC = 2
NS = 16
NW = NC * NS
L = 16

GROUPS = N // L
GPW = GROUPS // NW
PPW = GPW * L
TSEG = 64
NSEG = T // TSEG
UNROLL = 8
CUT = 23.03
TSF = 32
CUTF = 20.7


def _make_sc_body(nseg):
  def _sc_body(zt_hbm, r_hbm, ii_hbm, jj_hbm, num_hbm, s_hbm,
               ii_all, jj_all, r_all, ridx,
               rows, num0_st, num1_st, s_st, sem):
    wid = lax.axis_index("s") * NC + lax.axis_index("c")
    lanes = lax.iota(jnp.int32, L)
    base = wid * PPW
    pltpu.sync_copy(ii_hbm.at[pl.ds(base, PPW)], ii_all)
    pltpu.sync_copy(jj_hbm.at[pl.ds(base, PPW)], jj_all)
    pltpu.sync_copy(r_hbm.at[pl.ds(base, PPW)], r_all)

    def group_body(gl, carry0):
      ii = ii_all[pl.ds(gl * L, L)]
      jj = jj_all[pl.ds(gl * L, L)]
      r_v = r_all[pl.ds(gl * L, L)]
      d = jnp.exp(-r_v)
      r_min = jnp.min(r_v)
      lanes_f = lanes.astype(jnp.float32)
      step = r_min * float(TSEG)
      n_segs = jnp.sum((lanes_f * step < CUT).astype(jnp.int32))
      if nseg > L:
        n_segs = n_segs + jnp.sum(
            ((lanes_f + float(L)) * step < CUT).astype(jnp.int32))
      n_segs = jnp.minimum(n_segs, nseg)

      for b in range(B):
        row_i = (ii + b * D) * nseg
        row_j = (jj + b * D) * nseg

        def seg_body(s, seg_carry):
          w, acc, ssum = seg_carry
          ridx[pl.ds(0, L)] = row_i + (nseg - 1 - s)
          ridx[pl.ds(L, L)] = row_j + (nseg - 1 - s)
          pltpu.async_copy(zt_hbm.at[ridx], rows, sem).wait()

          def t_chunk(c, ch_carry):
            w, acc, ssum, tvec = ch_carry
            for _ in range(UNROLL):
              zi = plsc.load_gather(rows, [lanes, tvec])
              zj = plsc.load_gather(rows, [lanes + L, tvec])
              acc = acc + zi * zj * w
              ssum = ssum + w
              w = w * d
              tvec = tvec - 1
            return w, acc, ssum, tvec

          init = (w, acc, ssum, jnp.full((L,), TSEG - 1, jnp.int32))
          res = lax.fori_loop(0, TSEG // UNROLL, t_chunk, init)
          return res[0], res[1], res[2]

        init = (jnp.ones((L,), jnp.float32),
                jnp.zeros((L,), jnp.float32),
                jnp.zeros((L,), jnp.float32))
        _, acc, ssum = lax.fori_loop(0, n_segs, seg_body, init)

        if b == 0:
          num0_st[pl.ds(gl * L, L)] = acc
          s_st[pl.ds(gl * L, L)] = ssum
        else:
          num1_st[pl.ds(gl * L, L)] = acc
      return carry0

    lax.fori_loop(0, GPW, group_body, None)

    pltpu.sync_copy(num0_st, num_hbm.at[0, pl.ds(base, PPW)])
    pltpu.sync_copy(num1_st, num_hbm.at[1, pl.ds(base, PPW)])
    pltpu.sync_copy(s_st, s_hbm.at[pl.ds(base, PPW)])

  return _sc_body


def _make_sc_call(nseg):
  return functools.partial(
      pl.kernel,
      mesh=plsc.VectorSubcoreMesh(core_axis_name="c", subcore_axis_name="s"),
      compiler_params=pltpu.CompilerParams(
          use_tc_tiling_on_sc=False, needs_layout_passes=False),
      out_type=[jax.ShapeDtypeStruct((B, N), jnp.float32),
                jax.ShapeDtypeStruct((N,), jnp.float32)],
      scratch_types=[
          pltpu.VMEM((PPW,), jnp.int32),
          pltpu.VMEM((PPW,), jnp.int32),
          pltpu.VMEM((PPW,), jnp.float32),
          pltpu.VMEM((2 * L,), jnp.int32),
          pltpu.VMEM((2 * L, TSEG), jnp.float32),
          pltpu.VMEM((PPW,), jnp.float32),
          pltpu.VMEM((PPW,), jnp.float32),
          pltpu.VMEM((PPW,), jnp.float32),
          pltpu.SemaphoreType.DMA,
      ],
  )(_make_sc_body(nseg))


_sc_call_general = _make_sc_call(NSEG)


def _fast_body(zt_hbm, r_hbm, ii_hbm, jj_hbm, num_hbm, s_hbm,
               ii_all, jj_all, r_all, ridx0, ridx1, rows0, rows1,
               num0_st, num1_st, s_st, sem0, sem1):
  wid = lax.axis_index("s") * NC + lax.axis_index("c")
  lanes = lax.iota(jnp.int32, L)
  base = wid * PPW
  pltpu.sync_copy(ii_hbm.at[pl.ds(base, PPW)], ii_all)
  pltpu.sync_copy(jj_hbm.at[pl.ds(base, PPW)], jj_all)
  pltpu.sync_copy(r_hbm.at[pl.ds(base, PPW)], r_all)

  ridx = (ridx0, ridx1)
  rows = (rows0, rows1)
  sems = (sem0, sem1)
  NU = GPW * B

  def issue(u, slot):
    gl, b = u // B, u % B
    ii = ii_all[pl.ds(gl * L, L)]
    jj = jj_all[pl.ds(gl * L, L)]
    ridx[slot][pl.ds(0, L)] = ii + b * D
    ridx[slot][pl.ds(L, L)] = jj + b * D
    return pltpu.async_copy(zt_hbm.at[ridx[slot]], rows[slot], sems[slot])

  def compute(u, slot):
    gl, b = u // B, u % B
    d = jnp.exp(-r_all[pl.ds(gl * L, L)])
    rw = rows[slot]

    def t_chunk(c, ch_carry):
      w, acc, ssum, tvec = ch_carry
      for _ in range(UNROLL):
        zi = plsc.load_gather(rw, [lanes, tvec])
        zj = plsc.load_gather(rw, [lanes + L, tvec])
        acc = acc + zi * zj * w
        ssum = ssum + w
        w = w * d
        tvec = tvec - 1
      return w, acc, ssum, tvec

    init = (jnp.ones((L,), jnp.float32),
            jnp.zeros((L,), jnp.float32),
            jnp.zeros((L,), jnp.float32),
            jnp.full((L,), TSF - 1, jnp.int32))
    res = lax.fori_loop(0, TSF // UNROLL, t_chunk, init)
    acc, ssum = res[1], res[2]
    if b == 0:
      num0_st[pl.ds(gl * L, L)] = acc
      s_st[pl.ds(gl * L, L)] = ssum
    else:
      num1_st[pl.ds(gl * L, L)] = acc

  h = issue(0, 0)
  for u in range(NU):
    slot = u % 2
    nh = issue(u + 1, 1 - slot) if u + 1 < NU else None
    h.wait()
    compute(u, slot)
    h = nh

  pltpu.sync_copy(num0_st, num_hbm.at[0, pl.ds(base, PPW)])
  pltpu.sync_copy(num1_st, num_hbm.at[1, pl.ds(base, PPW)])
  pltpu.sync_copy(s_st, s_hbm.at[pl.ds(base, PPW)])


_sc_call_fast = functools.partial(
    pl.kernel,
    mesh=plsc.VectorSubcoreMesh(core_axis_name="c", subcore_axis_name="s"),
    compiler_params=pltpu.CompilerParams(
        use_tc_tiling_on_sc=False, needs_layout_passes=False),
    out_type=[jax.ShapeDtypeStruct((B, N), jnp.float32),
              jax.ShapeDtypeStruct((N,), jnp.float32)],
    scratch_types=[
        pltpu.VMEM((PPW,), jnp.int32),
        pltpu.VMEM((PPW,), jnp.int32),
        pltpu.VMEM((PPW,), jnp.float32),
        pltpu.VMEM((2 * L,), jnp.int32),
        pltpu.VMEM((2 * L,), jnp.int32),
        pltpu.VMEM((2 * L, TSF), jnp.float32),
        pltpu.VMEM((2 * L, TSF), jnp.float32),
        pltpu.VMEM((PPW,), jnp.float32),
        pltpu.VMEM((PPW,), jnp.float32),
        pltpu.VMEM((PPW,), jnp.float32),
        pltpu.SemaphoreType.DMA,
        pltpu.SemaphoreType.DMA,
    ],
)(_fast_body)


def _general(z_hist, r, ii, jj):
  zt = jnp.transpose(z_hist, (0, 2, 1)).reshape(B * D * NSEG, TSEG)
  num, s = _sc_call_general(zt, r, ii, jj)
  return num / jnp.sqrt(s + EPS)[None, :]


def _fast(z_hist, r, ii, jj):
  zt = jnp.transpose(z_hist[:, T - TSF:, :], (0, 2, 1)).reshape(B * D, TSF)
  num, s = _sc_call_fast(zt, r, ii, jj)
  return num / jnp.sqrt(s + EPS)[None, :]


@jax.jit
def kernel(z_hist, decay_rates, idx_i, idx_j):
  r = jax.nn.softplus(decay_rates)
  ii = idx_i.astype(jnp.int32)
  jj = idx_j.astype(jnp.int32)
  slab_ok = jnp.min(r) * float(TSF) >= CUTF
  return lax.cond(slab_ok,
                  lambda: _fast(z_hist, r, ii, jj),
                  lambda: _general(z_hist, r, ii, jj))


# speedup vs baseline: 2.1119x; 1.1205x over previous
import functools

import jax
import jax.numpy as jnp
from jax import lax
from jax.experimental import pallas as pl
from jax.experimental.pallas import tpu as pltpu
from jax.experimental.pallas import tpu_sc as plsc

D = 2048
T = 2048
B = 2
N = 8192
EPS = 1e-8

NC = 2
NS = 16
NW = NC * NS
L = 16

GROUPS = N // L
GPW = GROUPS // NW
PPW = GPW * L
TSEG = 64
NSEG = T // TSEG
UNROLL = 8
CUT = 23.03
TSF = 16
CUTF = 11.0


def _make_sc_body(nseg):
  def _sc_body(zt_hbm, r_hbm, ii_hbm, jj_hbm, num_hbm, s_hbm,
               ii_all, jj_all, r_all, ridx,
               rows, num0_st, num1_st, s_st, sem):
    wid = lax.axis_index("s") * NC + lax.axis_index("c")
    lanes = lax.iota(jnp.int32, L)
    base = wid * PPW
    pltpu.sync_copy(ii_hbm.at[pl.ds(base, PPW)], ii_all)
    pltpu.sync_copy(jj_hbm.at[pl.ds(base, PPW)], jj_all)
    pltpu.sync_copy(r_hbm.at[pl.ds(base, PPW)], r_all)

    def group_body(gl, carry0):
      ii = ii_all[pl.ds(gl * L, L)]
      jj = jj_all[pl.ds(gl * L, L)]
      r_v = r_all[pl.ds(gl * L, L)]
      d = jnp.exp(-r_v)
      r_min = jnp.min(r_v)
      lanes_f = lanes.astype(jnp.float32)
      step = r_min * float(TSEG)
      n_segs = jnp.sum((lanes_f * step < CUT).astype(jnp.int32))
      if nseg > L:
        n_segs = n_segs + jnp.sum(
            ((lanes_f + float(L)) * step < CUT).astype(jnp.int32))
      n_segs = jnp.minimum(n_segs, nseg)

      for b in range(B):
        row_i = (ii + b * D) * nseg
        row_j = (jj + b * D) * nseg

        def seg_body(s, seg_carry):
          w, acc, ssum = seg_carry
          ridx[pl.ds(0, L)] = row_i + (nseg - 1 - s)
          ridx[pl.ds(L, L)] = row_j + (nseg - 1 - s)
          pltpu.async_copy(zt_hbm.at[ridx], rows, sem).wait()

          def t_chunk(c, ch_carry):
            w, acc, ssum, tvec = ch_carry
            for _ in range(UNROLL):
              zi = plsc.load_gather(rows, [lanes, tvec])
              zj = plsc.load_gather(rows, [lanes + L, tvec])
              acc = acc + zi * zj * w
              ssum = ssum + w
              w = w * d
              tvec = tvec - 1
            return w, acc, ssum, tvec

          init = (w, acc, ssum, jnp.full((L,), TSEG - 1, jnp.int32))
          res = lax.fori_loop(0, TSEG // UNROLL, t_chunk, init)
          return res[0], res[1], res[2]

        init = (jnp.ones((L,), jnp.float32),
                jnp.zeros((L,), jnp.float32),
                jnp.zeros((L,), jnp.float32))
        _, acc, ssum = lax.fori_loop(0, n_segs, seg_body, init)

        if b == 0:
          num0_st[pl.ds(gl * L, L)] = acc
          s_st[pl.ds(gl * L, L)] = ssum
        else:
          num1_st[pl.ds(gl * L, L)] = acc
      return carry0

    lax.fori_loop(0, GPW, group_body, None)

    pltpu.sync_copy(num0_st, num_hbm.at[0, pl.ds(base, PPW)])
    pltpu.sync_copy(num1_st, num_hbm.at[1, pl.ds(base, PPW)])
    pltpu.sync_copy(s_st, s_hbm.at[pl.ds(base, PPW)])

  return _sc_body


def _make_sc_call(nseg):
  return functools.partial(
      pl.kernel,
      mesh=plsc.VectorSubcoreMesh(core_axis_name="c", subcore_axis_name="s"),
      compiler_params=pltpu.CompilerParams(
          use_tc_tiling_on_sc=False, needs_layout_passes=False),
      out_type=[jax.ShapeDtypeStruct((B, N), jnp.float32),
                jax.ShapeDtypeStruct((N,), jnp.float32)],
      scratch_types=[
          pltpu.VMEM((PPW,), jnp.int32),
          pltpu.VMEM((PPW,), jnp.int32),
          pltpu.VMEM((PPW,), jnp.float32),
          pltpu.VMEM((2 * L,), jnp.int32),
          pltpu.VMEM((2 * L, TSEG), jnp.float32),
          pltpu.VMEM((PPW,), jnp.float32),
          pltpu.VMEM((PPW,), jnp.float32),
          pltpu.VMEM((PPW,), jnp.float32),
          pltpu.SemaphoreType.DMA,
      ],
  )(_make_sc_body(nseg))


_sc_call_general = _make_sc_call(NSEG)


def _fast_body(zt_hbm, r_hbm, ii_hbm, jj_hbm, num_hbm, s_hbm,
               ii_all, jj_all, r_all, ridx0, ridx1, rows0, rows1,
               num0_st, num1_st, s_st, sem0, sem1):
  wid = lax.axis_index("s") * NC + lax.axis_index("c")
  lanes = lax.iota(jnp.int32, L)
  base = wid * PPW
  pltpu.sync_copy(ii_hbm.at[pl.ds(base, PPW)], ii_all)
  pltpu.sync_copy(jj_hbm.at[pl.ds(base, PPW)], jj_all)
  pltpu.sync_copy(r_hbm.at[pl.ds(base, PPW)], r_all)

  ridx = (ridx0, ridx1)
  rows = (rows0, rows1)
  sems = (sem0, sem1)
  NU = GPW * B

  def issue(u, slot):
    gl, b = u // B, u % B
    ii = ii_all[pl.ds(gl * L, L)]
    jj = jj_all[pl.ds(gl * L, L)]
    ridx[slot][pl.ds(0, L)] = ii + b * D
    ridx[slot][pl.ds(L, L)] = jj + b * D
    return pltpu.async_copy(zt_hbm.at[ridx[slot]], rows[slot], sems[slot])

  def compute(u, slot):
    gl, b = u // B, u % B
    d = jnp.exp(-r_all[pl.ds(gl * L, L)])
    rw = rows[slot]

    def t_chunk(c, ch_carry):
      w, acc, ssum, tvec = ch_carry
      for _ in range(UNROLL):
        zi = plsc.load_gather(rw, [lanes, tvec])
        zj = plsc.load_gather(rw, [lanes + L, tvec])
        acc = acc + zi * zj * w
        ssum = ssum + w
        w = w * d
        tvec = tvec - 1
      return w, acc, ssum, tvec

    init = (jnp.ones((L,), jnp.float32),
            jnp.zeros((L,), jnp.float32),
            jnp.zeros((L,), jnp.float32),
            jnp.full((L,), TSF - 1, jnp.int32))
    res = lax.fori_loop(0, TSF // UNROLL, t_chunk, init)
    acc, ssum = res[1], res[2]
    if b == 0:
      num0_st[pl.ds(gl * L, L)] = acc
      s_st[pl.ds(gl * L, L)] = ssum
    else:
      num1_st[pl.ds(gl * L, L)] = acc

  h = issue(0, 0)
  for u in range(NU):
    slot = u % 2
    nh = issue(u + 1, 1 - slot) if u + 1 < NU else None
    h.wait()
    compute(u, slot)
    h = nh

  pltpu.sync_copy(num0_st, num_hbm.at[0, pl.ds(base, PPW)])
  pltpu.sync_copy(num1_st, num_hbm.at[1, pl.ds(base, PPW)])
  pltpu.sync_copy(s_st, s_hbm.at[pl.ds(base, PPW)])


_sc_call_fast = functools.partial(
    pl.kernel,
    mesh=plsc.VectorSubcoreMesh(core_axis_name="c", subcore_axis_name="s"),
    compiler_params=pltpu.CompilerParams(
        use_tc_tiling_on_sc=False, needs_layout_passes=False),
    out_type=[jax.ShapeDtypeStruct((B, N), jnp.float32),
              jax.ShapeDtypeStruct((N,), jnp.float32)],
    scratch_types=[
        pltpu.VMEM((PPW,), jnp.int32),
        pltpu.VMEM((PPW,), jnp.int32),
        pltpu.VMEM((PPW,), jnp.float32),
        pltpu.VMEM((2 * L,), jnp.int32),
        pltpu.VMEM((2 * L,), jnp.int32),
        pltpu.VMEM((2 * L, TSF), jnp.float32),
        pltpu.VMEM((2 * L, TSF), jnp.float32),
        pltpu.VMEM((PPW,), jnp.float32),
        pltpu.VMEM((PPW,), jnp.float32),
        pltpu.VMEM((PPW,), jnp.float32),
        pltpu.SemaphoreType.DMA,
        pltpu.SemaphoreType.DMA,
    ],
)(_fast_body)


def _general(z_hist, r, ii, jj):
  zt = jnp.transpose(z_hist, (0, 2, 1)).reshape(B * D * NSEG, TSEG)
  num, s = _sc_call_general(zt, r, ii, jj)
  return num / jnp.sqrt(s + EPS)[None, :]


def _fast(z_hist, r, ii, jj):
  zt = jnp.transpose(z_hist[:, T - TSF:, :], (0, 2, 1)).reshape(B * D, TSF)
  num, s = _sc_call_fast(zt, r, ii, jj)
  return num / jnp.sqrt(s + EPS)[None, :]


@jax.jit
def kernel(z_hist, decay_rates, idx_i, idx_j):
  r = jax.nn.softplus(decay_rates)
  ii = idx_i.astype(jnp.int32)
  jj = idx_j.astype(jnp.int32)
  slab_ok = jnp.min(r) * float(TSF) >= CUTF
  return lax.cond(slab_ok,
                  lambda: _fast(z_hist, r, ii, jj),
                  lambda: _general(z_hist, r, ii, jj))
